# baseline (speedup 1.0000x reference)
import functools

import jax
import jax.numpy as jnp
from jax import lax
from jax.experimental import pallas as pl
from jax.experimental.pallas import tpu as pltpu
from jax.experimental.pallas import tpu_sc as plsc

NLAB = 512
MINI = 0.2
MAXI = 0.8
NC, NS, L = 2, 16, 16
NW = NC * NS
ZDIM, YDIM, XDIM = 32, 512, 512
R = 16
NCHUNK = YDIM // R
NBUF = 4
U = 8
GROUPS = XDIM // (U * L)

_mesh = plsc.VectorSubcoreMesh(
    core_axis_name="c", subcore_axis_name="s", num_cores=NC, num_subcores=NS
)
_params = pltpu.CompilerParams(
    needs_layout_passes=False, use_tc_tiling_on_sc=True
)


def _wid():
    return lax.axis_index("s") * NC + lax.axis_index("c")


@functools.partial(
    pl.kernel,
    out_type=jax.ShapeDtypeStruct((NW * 2 * NLAB,), jnp.float32),
    mesh=_mesh,
    compiler_params=_params,
    scratch_types=[
        pltpu.VMEM((NBUF, R, XDIM), jnp.int32),
        pltpu.VMEM((NBUF, R, XDIM), jnp.float32),
        pltpu.VMEM((NLAB,), jnp.float32),
        pltpu.VMEM((NLAB,), jnp.float32),
        pltpu.VMEM((2 * NLAB,), jnp.float32),
        [pltpu.SemaphoreType.DMA] * NBUF,
        [pltpu.SemaphoreType.DMA] * NBUF,
    ],
)
def _pass1(lab_hbm, int_hbm, tbl_hbm, lab_v, int_v, sums_v, cnts_v, row_v,
           sem_lab, sem_int):
    wid = _wid()
    zero16 = jnp.zeros((L,), jnp.float32)
    ones16 = jnp.ones((L,), jnp.float32)

    def zbody(i, _):
        sums_v[pl.ds(i * L, L)] = zero16
        cnts_v[pl.ds(i * L, L)] = zero16
        return 0

    lax.fori_loop(0, NLAB // L, zbody, 0)

    for b in range(NBUF):
        r0 = b * R
        pltpu.async_copy(
            lab_hbm.at[wid, pl.ds(r0, R)], lab_v.at[b], sem_lab[b]
        )
        pltpu.async_copy(
            int_hbm.at[wid, pl.ds(r0, R)], int_v.at[b], sem_int[b]
        )

    def pair_body(c2, _):
        for b in range(NBUF):
            c = c2 * NBUF + b
            r0 = c * R
            pltpu.make_async_copy(
                lab_hbm.at[wid, pl.ds(r0, R)], lab_v.at[b], sem_lab[b]
            ).wait()
            pltpu.make_async_copy(
                int_hbm.at[wid, pl.ds(r0, R)], int_v.at[b], sem_int[b]
            ).wait()

            @plsc.parallel_loop(0, R)
            def row_body(r):
                for g in range(GROUPS):
                    g0 = g * U * L
                    labs = [
                        lab_v[b, r, pl.ds(g0 + u * L, L)] for u in range(U)
                    ]
                    vals = [
                        int_v[b, r, pl.ds(g0 + u * L, L)] for u in range(U)
                    ]
                    idxs = labs
                    for u in range(U):
                        plsc.addupdate_scatter(sums_v, [idxs[u]], vals[u])
                    for u in range(U):
                        plsc.addupdate_scatter(cnts_v, [idxs[u]], ones16)

            @pl.when(c + NBUF < NCHUNK)
            def _():
                r2 = (c + NBUF) * R
                pltpu.async_copy(
                    lab_hbm.at[wid, pl.ds(r2, R)], lab_v.at[b], sem_lab[b]
                )
                pltpu.async_copy(
                    int_hbm.at[wid, pl.ds(r2, R)], int_v.at[b], sem_int[b]
                )
        return 0

    lax.fori_loop(0, NCHUNK // NBUF, pair_body, 0)

    def red_body(j, _):
        jl = j * L
        row_v[pl.ds(jl, L)] = sums_v[pl.ds(jl, L)]
        row_v[pl.ds(NLAB + jl, L)] = cnts_v[pl.ds(jl, L)]
        return 0

    lax.fori_loop(0, NLAB // L, red_body, 0)
    pltpu.sync_copy(row_v, tbl_hbm.at[pl.ds(wid * 2 * NLAB, 2 * NLAB)])


@functools.partial(
    pl.kernel,
    out_type=jax.ShapeDtypeStruct((ZDIM, YDIM, XDIM), jnp.int32),
    mesh=_mesh,
    compiler_params=_params,
    scratch_types=[
        pltpu.VMEM((NW * 2 * NLAB,), jnp.float32),
        pltpu.VMEM((NLAB,), jnp.int32),
        pltpu.VMEM((NBUF, R, XDIM), jnp.int32),
        pltpu.VMEM((NBUF, R, XDIM), jnp.int32),
        [pltpu.SemaphoreType.DMA] * NBUF,
        [pltpu.SemaphoreType.DMA] * NBUF,
    ],
)
def _pass2(lab_hbm, tbl_hbm, out_hbm, tbl_v, remap_v, lab_v, out_v,
           sem_lab, sem_out):
    wid = _wid()
    zero16 = jnp.zeros((L,), jnp.float32)
    iota16 = lax.iota(jnp.int32, L)

    pltpu.sync_copy(tbl_hbm, tbl_v)

    def rbody(j, _):
        jl = j * L
        s = zero16
        c = zero16
        for w in range(NW):
            s = s + tbl_v[pl.ds(w * 2 * NLAB + jl, L)]
            c = c + tbl_v[pl.ds(w * 2 * NLAB + NLAB + jl, L)]
        mean = s / jnp.maximum(c, 1.0)
        ids = iota16 + jl
        bad = ((mean < MINI) | (mean > MAXI)) & (ids != 0) & (c > 0.0)
        remap_v[pl.ds(jl, L)] = jnp.where(bad, 0, ids)
        return 0

    lax.fori_loop(0, NLAB // L, rbody, 0)

    for b in range(NBUF):
        r0 = b * R
        pltpu.async_copy(
            lab_hbm.at[wid, pl.ds(r0, R)], lab_v.at[b], sem_lab[b]
        )

    def pair_body(c2, _):
        for b in range(NBUF):
            c = c2 * NBUF + b
            r0 = c * R
            pltpu.make_async_copy(
                lab_hbm.at[wid, pl.ds(r0, R)], lab_v.at[b], sem_lab[b]
            ).wait()

            @pl.when(c >= NBUF)
            def _():
                rp = r0 - NBUF * R
                pltpu.make_async_copy(
                    out_v.at[b], out_hbm.at[wid, pl.ds(rp, R)], sem_out[b]
                ).wait()

            @plsc.parallel_loop(0, R)
            def row_body(r):
                for g in range(GROUPS):
                    g0 = g * U * L
                    labs = [
                        lab_v[b, r, pl.ds(g0 + u * L, L)] for u in range(U)
                    ]
                    news = [plsc.load_gather(remap_v, [lab]) for lab in labs]
                    for u in range(U):
                        out_v[b, r, pl.ds(g0 + u * L, L)] = news[u]
            pltpu.async_copy(
                out_v.at[b], out_hbm.at[wid, pl.ds(r0, R)], sem_out[b]
            )

            @pl.when(c + NBUF < NCHUNK)
            def _():
                r2 = (c + NBUF) * R
                pltpu.async_copy(
                    lab_hbm.at[wid, pl.ds(r2, R)], lab_v.at[b], sem_lab[b]
                )
        return 0

    lax.fori_loop(0, NCHUNK // NBUF, pair_body, 0)

    for b in range(NBUF):
        r0 = (NCHUNK - NBUF + b) * R
        pltpu.make_async_copy(
            out_v.at[b], out_hbm.at[wid, pl.ds(r0, R)], sem_out[b]
        ).wait()


def kernel(label_image, intensity_image):
    tbl = _pass1(label_image, intensity_image)
    return _pass2(label_image, tbl)


# speedup vs baseline: 1.0171x; 1.0171x over previous
import functools

import jax
import jax.numpy as jnp
from jax import lax
from jax.experimental import pallas as pl
from jax.experimental.pallas import tpu as pltpu
from jax.experimental.pallas import tpu_sc as plsc

NLAB = 512
MINI = 0.2
MAXI = 0.8
NC, NS, L = 2, 16, 16
NW = NC * NS
ZDIM, YDIM, XDIM = 32, 512, 512
R = 32
NCHUNK = YDIM // R
U = 8
GROUPS = XDIM // (U * L)
TBL = NW * 2 * NLAB
FLAG_OFF = TBL
MAGIC = 1234567.0

_mesh = plsc.VectorSubcoreMesh(
    core_axis_name="c", subcore_axis_name="s", num_cores=NC, num_subcores=NS
)
_params = pltpu.CompilerParams(
    needs_layout_passes=False, use_tc_tiling_on_sc=True
)


@functools.partial(
    pl.kernel,
    out_type=(
        jax.ShapeDtypeStruct((ZDIM, YDIM, XDIM), jnp.int32),
        jax.ShapeDtypeStruct((TBL + 2 * L,), jnp.float32),
    ),
    mesh=_mesh,
    compiler_params=_params,
    scratch_types=[
        pltpu.VMEM((2, R, XDIM), jnp.int32),
        pltpu.VMEM((2, R, XDIM), jnp.float32),
        pltpu.VMEM((2, R, XDIM), jnp.int32),
        pltpu.VMEM((8 * 2 * NLAB,), jnp.float32),
        pltpu.VMEM((NLAB,), jnp.float32),
        pltpu.VMEM((NLAB,), jnp.float32),
        pltpu.VMEM((2 * NLAB,), jnp.float32),
        pltpu.VMEM((NLAB,), jnp.int32),
        pltpu.VMEM((2 * L,), jnp.float32),
        [pltpu.SemaphoreType.DMA] * 2,
        [pltpu.SemaphoreType.DMA] * 2,
        [pltpu.SemaphoreType.DMA] * 2,
    ],
)
def _fused(lab_hbm, int_hbm, out_hbm, tbl_hbm, lab_v, int_v, out_v, piece_v,
           sums_v, cnts_v, row_v, remap_v, flag_v, sem_lab, sem_int, sem_out):
    cid = lax.axis_index("c")
    sid = lax.axis_index("s")
    wid = sid * NC + cid
    zero16 = jnp.zeros((L,), jnp.float32)
    ones16 = jnp.ones((L,), jnp.float32)
    magic16 = jnp.full((L,), MAGIC, jnp.float32)
    iota16 = lax.iota(jnp.int32, L)

    @pl.when(sid == 0)
    def _():
        flag_v[pl.ds(0, L)] = zero16
        pltpu.sync_copy(
            flag_v.at[pl.ds(0, L)], tbl_hbm.at[pl.ds(FLAG_OFF + cid * L, L)]
        )

    def zbody(i, _):
        sums_v[pl.ds(i * L, L)] = zero16
        cnts_v[pl.ds(i * L, L)] = zero16
        return 0

    lax.fori_loop(0, NLAB // L, zbody, 0)

    for b in range(2):
        r0 = b * R
        pltpu.async_copy(lab_hbm.at[wid, pl.ds(r0, R)], lab_v.at[b], sem_lab[b])
        pltpu.async_copy(int_hbm.at[wid, pl.ds(r0, R)], int_v.at[b], sem_int[b])

    def p1_body(c2, _):
        for b in range(2):
            c = c2 * 2 + b
            r0 = c * R
            pltpu.make_async_copy(
                lab_hbm.at[wid, pl.ds(r0, R)], lab_v.at[b], sem_lab[b]
            ).wait()
            pltpu.make_async_copy(
                int_hbm.at[wid, pl.ds(r0, R)], int_v.at[b], sem_int[b]
            ).wait()

            @plsc.parallel_loop(0, R)
            def row_body(r):
                for g in range(GROUPS):
                    g0 = g * U * L
                    labs = [
                        lab_v[b, r, pl.ds(g0 + u * L, L)] for u in range(U)
                    ]
                    vals = [
                        int_v[b, r, pl.ds(g0 + u * L, L)] for u in range(U)
                    ]
                    for u in range(U):
                        plsc.addupdate_scatter(sums_v, [labs[u]], vals[u])
                    for u in range(U):
                        plsc.addupdate_scatter(cnts_v, [labs[u]], ones16)

            @pl.when(c + 2 < NCHUNK)
            def _():
                r2 = (c + 2) * R
                pltpu.async_copy(
                    lab_hbm.at[wid, pl.ds(r2, R)], lab_v.at[b], sem_lab[b]
                )
                pltpu.async_copy(
                    int_hbm.at[wid, pl.ds(r2, R)], int_v.at[b], sem_int[b]
                )
        return 0

    lax.fori_loop(0, NCHUNK // 2, p1_body, 0)

    def pack_body(j, _):
        jl = j * L
        row_v[pl.ds(jl, L)] = sums_v[pl.ds(jl, L)]
        row_v[pl.ds(NLAB + jl, L)] = cnts_v[pl.ds(jl, L)]
        return 0

    lax.fori_loop(0, NLAB // L, pack_body, 0)
    pltpu.sync_copy(row_v, tbl_hbm.at[pl.ds(wid * 2 * NLAB, 2 * NLAB)])

    plsc.subcore_barrier()

    @pl.when(sid == 0)
    def _():
        flag_v[pl.ds(0, L)] = magic16
        pltpu.sync_copy(
            flag_v.at[pl.ds(0, L)], tbl_hbm.at[pl.ds(FLAG_OFF + cid * L, L)]
        )

    for b in range(2):
        r0 = b * R
        pltpu.async_copy(lab_hbm.at[wid, pl.ds(r0, R)], lab_v.at[b], sem_lab[b])

    def poll_cond(done):
        return jnp.logical_not(done)

    def poll_body(done):
        pltpu.sync_copy(tbl_hbm.at[pl.ds(FLAG_OFF, 2 * L)], flag_v)
        f0 = flag_v[pl.ds(0, L)]
        f1 = flag_v[pl.ds(L, L)]
        return jnp.all((f0 == MAGIC) & (f1 == MAGIC))

    lax.while_loop(poll_cond, poll_body, False)

    for p in range(NW // 8):
        pltpu.sync_copy(
            tbl_hbm.at[pl.ds(p * 8 * 2 * NLAB, 8 * 2 * NLAB)], piece_v
        )

        def acc_body(j, _):
            jl = j * L
            s = zero16 if p == 0 else sums_v[pl.ds(jl, L)]
            c = zero16 if p == 0 else cnts_v[pl.ds(jl, L)]
            for w in range(8):
                s = s + piece_v[pl.ds(w * 2 * NLAB + jl, L)]
                c = c + piece_v[pl.ds(w * 2 * NLAB + NLAB + jl, L)]
            sums_v[pl.ds(jl, L)] = s
            cnts_v[pl.ds(jl, L)] = c
            return 0

        lax.fori_loop(0, NLAB // L, acc_body, 0)

    def rbody(j, _):
        jl = j * L
        s = sums_v[pl.ds(jl, L)]
        c = cnts_v[pl.ds(jl, L)]
        mean = s / jnp.maximum(c, 1.0)
        ids = iota16 + jl
        bad = ((mean < MINI) | (mean > MAXI)) & (ids != 0) & (c > 0.0)
        remap_v[pl.ds(jl, L)] = jnp.where(bad, 0, ids)
        return 0

    lax.fori_loop(0, NLAB // L, rbody, 0)

    def p2_body(c2, _):
        for b in range(2):
            c = c2 * 2 + b
            r0 = c * R
            pltpu.make_async_copy(
                lab_hbm.at[wid, pl.ds(r0, R)], lab_v.at[b], sem_lab[b]
            ).wait()

            @pl.when(c >= 2)
            def _():
                rp = r0 - 2 * R
                pltpu.make_async_copy(
                    out_v.at[b], out_hbm.at[wid, pl.ds(rp, R)], sem_out[b]
                ).wait()

            @plsc.parallel_loop(0, R)
            def row_body(r):
                for g in range(GROUPS):
                    g0 = g * U * L
                    labs = [
                        lab_v[b, r, pl.ds(g0 + u * L, L)] for u in range(U)
                    ]
                    news = [plsc.load_gather(remap_v, [lab]) for lab in labs]
                    for u in range(U):
                        out_v[b, r, pl.ds(g0 + u * L, L)] = news[u]

            pltpu.async_copy(
                out_v.at[b], out_hbm.at[wid, pl.ds(r0, R)], sem_out[b]
            )

            @pl.when(c + 2 < NCHUNK)
            def _():
                r2 = (c + 2) * R
                pltpu.async_copy(
                    lab_hbm.at[wid, pl.ds(r2, R)], lab_v.at[b], sem_lab[b]
                )
        return 0

    lax.fori_loop(0, NCHUNK // 2, p2_body, 0)

    for b in range(2):
        r0 = (NCHUNK - 2 + b) * R
        pltpu.make_async_copy(
            out_v.at[b], out_hbm.at[wid, pl.ds(r0, R)], sem_out[b]
        ).wait()


def kernel(label_image, intensity_image):
    out, _ = _fused(label_image, intensity_image)
    return out
